# baseline (speedup 1.0000x reference)
import functools

import jax
import jax.numpy as jnp
from jax import lax
from jax.experimental import pallas as pl
from jax.experimental.pallas import tpu as pltpu
from jax.experimental.pallas import tpu_sc as plsc

_NUM_F = 9
_NUM_T = 2
_CHUNK = 128
_L = 16
_NB = 5


@functools.lru_cache(maxsize=None)
def _build_kernel(n_pad: int, hidden: int):
    info = plsc.get_sparse_core_info()
    nc, ns = info.num_cores, info.num_subcores
    nw = nc * ns
    pair_chunks = n_pad // (ns * _CHUNK)
    n_chunks = pair_chunks // nc
    assert n_chunks % _NB == 0
    n_outer = n_chunks // _NB
    mesh = plsc.VectorSubcoreMesh(core_axis_name="c", subcore_axis_name="s")

    @functools.partial(
        pl.kernel,
        out_type=jax.ShapeDtypeStruct((n_pad, hidden), jnp.float32),
        mesh=mesh,
        scratch_types=[
            pltpu.VMEM((_NB, _NUM_F, _CHUNK), jnp.int32),
            pltpu.VMEM((_NB, _NUM_T, _CHUNK), jnp.int32),
            pltpu.VMEM((_NB, _CHUNK, hidden), jnp.float32),
            pltpu.SemaphoreType.DMA((_NB,)),
            pltpu.SemaphoreType.DMA((_NB,)),
            pltpu.SemaphoreType.DMA((_NB,)),
            pltpu.SemaphoreType.DMA((_NB,)),
        ],
    )
    def k(x_b, t0, t1, out, idx_v, idx_c, acc, s_i, s_g0, s_ga, s_w):
        c = lax.axis_index("c")
        s = lax.axis_index("s")
        cbase = s * pair_chunks + c

        def cid(j):
            return cbase + 2 * j

        def fire_idx(j, b):
            pltpu.async_copy(x_b.at[cid(j)], idx_v.at[b], s_i.at[b])

        def wait_idx(j, b):
            pltpu.make_async_copy(x_b.at[cid(j)], idx_v.at[b], s_i.at[b]).wait()

        def fire_g0(b):
            pltpu.async_copy(t0.at[idx_c.at[b, 0]], acc.at[b], s_g0.at[b])

        def wait_g0(b):
            pltpu.make_async_copy(t0.at[idx_c.at[b, 0]], acc.at[b], s_g0.at[b]).wait()

        def fire_adds(b):
            pltpu.async_copy(t1.at[idx_c.at[b, 1]], acc.at[b], s_ga.at[b], add=False)

        def wait_adds(b):
            pltpu.make_async_copy(
                t1.at[idx_c.at[b, 1]], acc.at[b], s_ga.at[b]
            ).wait()

        def fire_wb(j, b):
            pltpu.async_copy(
                acc.at[b], out.at[pl.ds(cid(j) * _CHUNK, _CHUNK), :], s_w.at[b]
            )

        def wait_wb(j, b):
            pltpu.make_async_copy(
                acc.at[b], out.at[pl.ds(cid(j) * _CHUNK, _CHUNK), :], s_w.at[b]
            ).wait()

        def combine_indices(b):
            for g in range(_CHUNK // _L):
                sl = pl.ds(g * _L, _L)
                x0 = idx_v[b, 0, sl]
                x1 = idx_v[b, 1, sl]
                x2 = idx_v[b, 2, sl]
                x3 = idx_v[b, 3, sl]
                x4 = idx_v[b, 4, sl]
                x5 = idx_v[b, 5, sl]
                x6 = idx_v[b, 6, sl]
                x7 = idx_v[b, 7, sl]
                x8 = idx_v[b, 8, sl]
                idx_c[b, 0, sl] = (x0 * 12 + x3) * 9 + x4
                idx_c[b, 1, sl] = (
                    (((x1 * 11 + x2) * 5 + x5) * 8 + x6) * 4 + x7 * 2 + x8
                )

        fire_idx(0, 0)
        fire_idx(1, 1)

        def outer_body(o, carry):
            for b in range(_NB):
                j = o * _NB + b
                if b in (3, 4):
                    @pl.when(o <= n_outer - 2)
                    def _():
                        fire_idx(j + 2, (b + 2) % _NB)
                else:
                    fire_idx(j + 2, (b + 2) % _NB)
                wait_idx(j, b)
                combine_indices(b)

                @pl.when(o >= 1)
                def _():
                    wait_wb(j - _NB, b)
                fire_g0(b)
                q = (b - 1) % _NB
                if b >= 1:
                    wait_g0(q)
                    fire_adds(q)
                else:
                    @pl.when(o >= 1)
                    def _():
                        wait_g0(q)
                        fire_adds(q)
                r = (b - 2) % _NB
                if b >= 2:
                    wait_adds(r)
                    fire_wb(j - 2, r)
                else:
                    @pl.when(o >= 1)
                    def _():
                        wait_adds(r)
                        fire_wb(j - 2, r)
            return carry

        lax.fori_loop(0, n_outer, outer_body, 0)

        last = n_chunks - 1
        b_last = _NB - 1
        b_prev = _NB - 2
        wait_g0(b_last)
        fire_adds(b_last)
        wait_adds(b_prev)
        fire_wb(last - 1, b_prev)
        wait_adds(b_last)
        fire_wb(last, b_last)
        for b in range(_NB):
            wait_wb(n_chunks - _NB + b, b)

    return k


def _outer_sum(*tables):
    h = tables[0].shape[1]
    acc = tables[0]
    for t in tables[1:]:
        acc = (acc[:, None, :] + t[None, :, :]).reshape(-1, h)
    return acc


def kernel(x, emb_0, emb_1, emb_2, emb_3, emb_4, emb_5, emb_6, emb_7, emb_8):
    n = x.shape[0]
    hidden = emb_0.shape[1]
    info = plsc.get_sparse_core_info()
    nw = info.num_cores * info.num_subcores
    block = nw * _CHUNK
    n_pad = ((n + block - 1) // block) * block

    xi = x.astype(jnp.int32)
    if n_pad != n:
        xi = jnp.pad(xi, ((0, n_pad - n), (0, 0)))
    x_b = jnp.transpose(xi.reshape(n_pad // _CHUNK, _CHUNK, _NUM_F), (0, 2, 1))

    t0 = _outer_sum(emb_0, emb_3, emb_4)
    t1 = _outer_sum(emb_1, emb_2, emb_5, emb_6, emb_7, emb_8)

    k = _build_kernel(n_pad, hidden)
    out = k(x_b, t0, t1)
    return out[:n]


# speedup vs baseline: 2.2450x; 2.2450x over previous
import functools

import jax
import jax.numpy as jnp
from jax import lax
from jax.experimental import pallas as pl
from jax.experimental.pallas import tpu as pltpu
from jax.experimental.pallas import tpu_sc as plsc

_NUM_F = 9
_NUM_T = 2
_CHUNK = 128
_L = 16
_NB = 5


@functools.lru_cache(maxsize=None)
def _build_kernel(n: int, hidden: int):
    info = plsc.get_sparse_core_info()
    nc, ns = info.num_cores, info.num_subcores
    nw = nc * ns
    slots = ((n + nw * _CHUNK - 1) // (nw * _CHUNK)) * nw
    real = (n + _CHUNK - 1) // _CHUNK
    full = n // _CHUNK
    tail = n - full * _CHUNK
    assert tail == 0 or tail % 8 == 0
    pair_chunks = slots // ns
    n_chunks = pair_chunks // nc
    assert n_chunks % _NB == 0
    n_outer = n_chunks // _NB
    mesh = plsc.VectorSubcoreMesh(core_axis_name="c", subcore_axis_name="s")

    @functools.partial(
        pl.kernel,
        out_type=jax.ShapeDtypeStruct((n, hidden), jnp.float32),
        mesh=mesh,
        scratch_types=[
            pltpu.VMEM((_NB, _NUM_F, _CHUNK), jnp.int32),
            pltpu.VMEM((_NB, _NUM_T, _CHUNK), jnp.int32),
            pltpu.VMEM((_NB, _CHUNK, hidden), jnp.float32),
            pltpu.SemaphoreType.DMA((_NB,)),
            pltpu.SemaphoreType.DMA((_NB,)),
            pltpu.SemaphoreType.DMA((_NB,)),
            pltpu.SemaphoreType.DMA((_NB,)),
        ],
    )
    def k(x_b, t0, t1, out, idx_v, idx_c, acc, s_i, s_g0, s_ga, s_w):
        c = lax.axis_index("c")
        s = lax.axis_index("s")
        cbase = s * pair_chunks + c

        def cid(j):
            return cbase + 2 * j

        def _when_active(j, go):
            @pl.when(cid(j) < real)
            def _():
                go()

        def fire_idx(j, b):
            _when_active(j, lambda: pltpu.make_async_copy(
                x_b.at[cid(j)], idx_v.at[b], s_i.at[b]).start())

        def wait_idx(j, b):
            _when_active(j, lambda: pltpu.make_async_copy(
                x_b.at[cid(j)], idx_v.at[b], s_i.at[b]).wait())

        def fire_g0(j, b):
            _when_active(j, lambda: pltpu.make_async_copy(
                t0.at[idx_c.at[b, 0]], acc.at[b], s_g0.at[b]).start())

        def wait_g0(j, b):
            _when_active(j, lambda: pltpu.make_async_copy(
                t0.at[idx_c.at[b, 0]], acc.at[b], s_g0.at[b]).wait())

        def fire_adds(j, b):
            _when_active(j, lambda: pltpu.make_async_copy(
                t1.at[idx_c.at[b, 1]], acc.at[b], s_ga.at[b]).start(add=True))

        def wait_adds(j, b):
            _when_active(j, lambda: pltpu.make_async_copy(
                t1.at[idx_c.at[b, 1]], acc.at[b], s_ga.at[b]).wait())

        def _wb_full(j, b):
            return pltpu.make_async_copy(
                acc.at[b], out.at[pl.ds(cid(j) * _CHUNK, _CHUNK), :], s_w.at[b]
            )

        def _wb_tail(b):
            return pltpu.make_async_copy(
                acc.at[b, pl.ds(0, tail), :],
                out.at[pl.ds(full * _CHUNK, tail), :],
                s_w.at[b],
            )

        def fire_wb(j, b):
            jg = cid(j)

            @pl.when(jg < full)
            def _():
                _wb_full(j, b).start()

            if tail:
                @pl.when(jg == full)
                def _():
                    _wb_tail(b).start()

        def wait_wb(j, b):
            jg = cid(j)

            @pl.when(jg < full)
            def _():
                _wb_full(j, b).wait()

            if tail:
                @pl.when(jg == full)
                def _():
                    _wb_tail(b).wait()

        def combine_indices(b):
            for g in range(_CHUNK // _L):
                sl = pl.ds(g * _L, _L)
                x0 = idx_v[b, 0, sl]
                x1 = idx_v[b, 1, sl]
                x2 = idx_v[b, 2, sl]
                x3 = idx_v[b, 3, sl]
                x4 = idx_v[b, 4, sl]
                x5 = idx_v[b, 5, sl]
                x6 = idx_v[b, 6, sl]
                x7 = idx_v[b, 7, sl]
                x8 = idx_v[b, 8, sl]
                idx_c[b, 0, sl] = (x0 * 12 + x3) * 9 + x4
                idx_c[b, 1, sl] = (
                    (((x1 * 11 + x2) * 5 + x5) * 8 + x6) * 4 + x7 * 2 + x8
                )

        fire_idx(0, 0)
        fire_idx(1, 1)

        def outer_body(o, carry):
            for b in range(_NB):
                j = o * _NB + b
                if b in (3, 4):
                    @pl.when(o <= n_outer - 2)
                    def _():
                        fire_idx(j + 2, (b + 2) % _NB)
                else:
                    fire_idx(j + 2, (b + 2) % _NB)
                wait_idx(j, b)
                combine_indices(b)

                @pl.when(o >= 1)
                def _():
                    wait_wb(j - _NB, b)
                fire_g0(j, b)
                q = (b - 1) % _NB
                if b >= 1:
                    wait_g0(j - 1, q)
                    fire_adds(j - 1, q)
                else:
                    @pl.when(o >= 1)
                    def _():
                        wait_g0(j - 1, q)
                        fire_adds(j - 1, q)
                r = (b - 2) % _NB
                if b >= 2:
                    wait_adds(j - 2, r)
                    fire_wb(j - 2, r)
                else:
                    @pl.when(o >= 1)
                    def _():
                        wait_adds(j - 2, r)
                        fire_wb(j - 2, r)
            return carry

        lax.fori_loop(0, n_outer, outer_body, 0)

        last = n_chunks - 1
        b_last = _NB - 1
        b_prev = _NB - 2
        wait_g0(last, b_last)
        fire_adds(last, b_last)
        wait_adds(last - 1, b_prev)
        fire_wb(last - 1, b_prev)
        wait_adds(last, b_last)
        fire_wb(last, b_last)
        for b in range(_NB):
            wait_wb(n_chunks - _NB + b, b)

    return k


def kernel(x, emb_0, emb_1, emb_2, emb_3, emb_4, emb_5, emb_6, emb_7, emb_8):
    n = x.shape[0]
    hidden = emb_0.shape[1]
    real = (n + _CHUNK - 1) // _CHUNK
    n_up = real * _CHUNK

    xi = x.astype(jnp.int32)
    if n_up != n:
        xi = jnp.pad(xi, ((0, n_up - n), (0, 0)))
    x_b = jnp.transpose(xi.reshape(real, _CHUNK, _NUM_F), (0, 2, 1))

    t0 = (
        emb_0[:, None, None, :] + emb_3[None, :, None, :] + emb_4[None, None, :, :]
    ).reshape(-1, hidden)
    t1 = (
        emb_1[:, None, None, None, None, None, :]
        + emb_2[None, :, None, None, None, None, :]
        + emb_5[None, None, :, None, None, None, :]
        + emb_6[None, None, None, :, None, None, :]
        + emb_7[None, None, None, None, :, None, :]
        + emb_8[None, None, None, None, None, :, :]
    ).reshape(-1, hidden)

    k = _build_kernel(n, hidden)
    return k(x_b, t0, t1)
